# Initial kernel scaffold; baseline (speedup 1.0000x reference)
import jax
import jax.numpy as jnp
from jax.experimental import pallas as pl


def kernel(q, k, v):
    raise NotImplementedError("write your pallas kernel here")


# speedup vs baseline: 1.1695x; 1.1695x over previous
import functools

import jax
import jax.numpy as jnp
from jax.experimental import pallas as pl
from jax.experimental.pallas import tpu as pltpu


def _attn_body(q_ref, k_ref, v_ref, o_ref, *, scale):
    q = q_ref[0]
    k = k_ref[0]
    v = v_ref[0]
    s = jax.lax.dot_general(
        q, k, (((1,), (1,)), ((), ())), preferred_element_type=jnp.float32
    )
    s = s * scale
    m = jnp.max(s, axis=-1, keepdims=True)
    p = jnp.exp(s - m)
    l = jnp.sum(p, axis=-1, keepdims=True)
    o = jax.lax.dot_general(
        p, v, (((1,), (0,)), ((), ())), preferred_element_type=jnp.float32
    )
    o_ref[0] = o / l


def kernel(q, k, v):
    B, H, S, D = q.shape
    bq = min(512, S)
    qf = q.reshape(B * H, S, D)
    kf = k.reshape(B * H, S, D)
    vf = v.reshape(B * H, S, D)
    scale = 1.0 / (D ** 0.5)

    out = pl.pallas_call(
        functools.partial(_attn_body, scale=scale),
        grid=(B * H, S // bq),
        in_specs=[
            pl.BlockSpec((1, bq, D), lambda h, i: (h, i, 0)),
            pl.BlockSpec((1, S, D), lambda h, i: (h, 0, 0)),
            pl.BlockSpec((1, S, D), lambda h, i: (h, 0, 0)),
        ],
        out_specs=pl.BlockSpec((1, bq, D), lambda h, i: (h, i, 0)),
        out_shape=jax.ShapeDtypeStruct((B * H, S, D), jnp.float32),
    )(qf, kf, vf)
    return out.reshape(B, H, S, D)
